# baseline (speedup 1.0000x reference)
import functools
import math

import jax
import jax.numpy as jnp
from jax import lax
from jax.experimental import pallas as pl
from jax.experimental.pallas import tpu as pltpu
from jax.experimental.pallas import tpu_sc as plsc

D_MODEL = 64
SCALE = math.sqrt(D_MODEL)
RPC = 8
SUB = 100
NBUF = 2


@functools.lru_cache(maxsize=None)
def _build(BATCH: int, SEQ: int, V: int):
    info = plsc.get_sparse_core_info()
    NC, NS, L = info.num_cores, info.num_subcores, info.num_lanes
    NW = NC * NS
    C = RPC * SEQ
    CP = 512
    NSUB = C // SUB
    H = C // 2
    assert C % SUB == 0 and H % 8 == 0 and SUB <= 128
    assert BATCH % (NW * RPC) == 0
    R = BATCH // (NW * RPC)
    assert R % NBUF == 0 and R > NBUF
    G = R // NBUF
    mesh = plsc.VectorSubcoreMesh(core_axis_name="c", subcore_axis_name="s")

    @functools.partial(
        pl.kernel,
        mesh=mesh,
        out_type=jax.ShapeDtypeStruct(
            (BATCH * SEQ // 2, 2 * D_MODEL), jnp.float32
        ),
        compiler_params=pltpu.CompilerParams(
            use_tc_tiling_on_sc=True, needs_layout_passes=False
        ),
        scratch_types=[
            pltpu.VMEM((NBUF, CP), jnp.int32),
            pltpu.VMEM((NBUF, NSUB, SUB), jnp.int32),
            pltpu.VMEM((NBUF, C, 2 * D_MODEL), jnp.float32),
            pltpu.SemaphoreType.DMA,
            pltpu.SemaphoreType.DMA,
        ],
    )
    def k(table_hbm, idx_hbm, out_hbm, idx_v, gidx_v, pair_v, gsem, osem):
        wid = lax.axis_index("s") * NC + lax.axis_index("c")
        j0 = wid * R
        lanes16 = jax.lax.iota(jnp.int32, 16)

        def load_build_gather(j, b):
            pltpu.sync_copy(idx_hbm.at[j0 + j], idx_v.at[b])
            bb = jnp.full((L,), b, jnp.int32)
            for h in range(NSUB):
                hh = jnp.full((L,), h, jnp.int32)
                for v in range((SUB + L - 1) // L):
                    ln = lanes16 + v * L
                    msk = ln < SUB
                    g = plsc.load_gather(idx_v, [bb, h * SUB + ln], mask=msk)
                    plsc.store_scatter(
                        gidx_v,
                        [bb, hh, ln],
                        jax.lax.shift_right_logical(g, 1),
                        mask=msk,
                    )
            for h in range(NSUB):
                pltpu.async_copy(
                    table_hbm.at[gidx_v.at[b, h]],
                    pair_v.at[b, pl.ds(h * SUB, SUB)],
                    gsem,
                )

        def wait_gather(b):
            for h in range(NSUB):
                pltpu.make_async_copy(
                    table_hbm.at[gidx_v.at[b, h]],
                    pair_v.at[b, pl.ds(h * SUB, SUB)],
                    gsem,
                ).wait()

        def drain_one_out(b):
            pltpu.make_async_copy(
                pair_v.at[b, pl.ds(0, H)], out_hbm.at[pl.ds(0, H)], osem
            ).wait()

        def select_scale(b):
            bb = jnp.full((L,), b, jnp.int32)

            def row_body(r, _):
                rr = jnp.full((L,), 0, jnp.int32) + r
                dr = jax.lax.shift_right_logical(rr, 1)
                dc = jax.lax.shift_left(rr & 1, 6)
                orig = plsc.load_gather(idx_v, [bb, rr])
                cb = jax.lax.shift_left(orig & 1, 6)
                for c in range(D_MODEL // L):
                    cl = cb + (c * L + lanes16)
                    vvec = plsc.load_gather(pair_v, [bb, rr, cl])
                    plsc.store_scatter(
                        pair_v, [bb, dr, dc + c * L + lanes16], vvec * SCALE
                    )
                return ()

            lax.fori_loop(0, C, row_body, ())

        def out_copies(j, b):
            pltpu.async_copy(
                pair_v.at[b, pl.ds(0, H)],
                out_hbm.at[pl.ds((j0 + j) * H, H)],
                osem,
            )

        for b in range(NBUF):
            load_build_gather(b, b)

        def group_body(g, _):
            for b in range(NBUF):
                j = g * NBUF + b
                wait_gather(b)
                select_scale(b)
                out_copies(j, b)
                bp = (b - 1) % NBUF
                cond = (g >= 1) if b == 0 else (g < G - 1)

                @pl.when(cond)
                def _():
                    drain_one_out(bp)
                    load_build_gather(j - 1 + NBUF, bp)

            return ()

        lax.fori_loop(0, G, group_body, ())

        for b in range(NBUF):
            drain_one_out(b)

    return k


def kernel(x, table):
    BATCH, SEQ = x.shape
    V = table.shape[0]
    table2 = table.reshape(V // 2, 2 * D_MODEL)
    idx = x.reshape(BATCH // RPC, RPC * SEQ).astype(jnp.int32)
    idx = jnp.pad(idx, ((0, 0), (0, 512 - RPC * SEQ)))
    out2 = _build(BATCH, SEQ, V)(table2, idx)
    return out2.reshape(BATCH, SEQ, D_MODEL)
